# Initial kernel scaffold; baseline (speedup 1.0000x reference)
import jax
import jax.numpy as jnp
from jax.experimental import pallas as pl


def kernel(x, edge_index, W1, b1, W2, b2):
    raise NotImplementedError("write your pallas kernel here")


# speedup vs baseline: 113.5414x; 113.5414x over previous
import jax
import jax.numpy as jnp
from jax import lax
from jax.experimental import pallas as pl
from jax.experimental.pallas import tpu as pltpu
from jax.experimental.pallas import tpu_sc as plsc

N_NODES = 100000
N_EDGES = 6400000
NPAD = 102400
ROWS = NPAD // 128
NC, NS = 2, 16
NW = NC * NS
CH = 32
CPT = 49
EPAD = NW * CPT * CH * 128
IDX_ROWS = EPAD // 128
SLICE = NPAD // NS

_mesh = plsc.VectorSubcoreMesh(core_axis_name="c", subcore_axis_name="s")


def _wid():
    return lax.axis_index("c") * NS + lax.axis_index("s")


def _zero_acc(zeros_hbm, acc):
    s = lax.axis_index("s")
    pltpu.sync_copy(zeros_hbm.at[pl.ds(s * SLICE, SLICE)],
                    acc.at[pl.ds(s * SLICE, SLICE)])


def _sc_deg(dst1d, ones, zeros):
    def body(dst_hbm, ones_hbm, zeros_hbm, out_hbm, idx_d, vals, acc):
        c = lax.axis_index("c")
        w = _wid()
        pltpu.sync_copy(ones_hbm, vals)
        _zero_acc(zeros_hbm, acc)
        plsc.subcore_barrier()

        def step(i, carry):
            eb = (w * CPT + i) * CH * 128
            pltpu.sync_copy(dst_hbm.at[pl.ds(eb, CH * 128)], idx_d)
            pltpu.sync_copy(vals, acc.at[idx_d], add=True)
            return carry

        lax.fori_loop(0, CPT, step, 0)
        plsc.subcore_barrier()

        @pl.when(lax.axis_index("s") == 0)
        def _():
            pltpu.sync_copy(acc, out_hbm.at[c])

    f = pl.kernel(
        body, mesh=_mesh,
        out_type=jax.ShapeDtypeStruct((NC, NPAD), jnp.float32),
        scratch_types=[
            pltpu.VMEM((CH * 128,), jnp.int32),
            pltpu.VMEM((CH * 128,), jnp.float32),
            pltpu.VMEM_SHARED((NPAD,), jnp.float32),
        ],
    )
    return f(dst1d, ones, zeros)


def _sc_agg2(src1d, dst1d, tab0, tab1, zeros):
    def body(src_hbm, dst_hbm, t0_hbm, t1_hbm, zeros_hbm, out0, out1,
             idx_s, idx_d, v0, v1, acc0, acc1):
        c = lax.axis_index("c")
        w = _wid()
        _zero_acc(zeros_hbm, acc0)
        _zero_acc(zeros_hbm, acc1)
        plsc.subcore_barrier()

        def step(i, carry):
            eb = (w * CPT + i) * CH * 128
            pltpu.sync_copy(src_hbm.at[pl.ds(eb, CH * 128)], idx_s)
            pltpu.sync_copy(dst_hbm.at[pl.ds(eb, CH * 128)], idx_d)
            pltpu.sync_copy(t0_hbm.at[idx_s], v0)
            pltpu.sync_copy(t1_hbm.at[idx_s], v1)
            pltpu.sync_copy(v0, acc0.at[idx_d], add=True)
            pltpu.sync_copy(v1, acc1.at[idx_d], add=True)
            return carry

        lax.fori_loop(0, CPT, step, 0)
        plsc.subcore_barrier()

        @pl.when(lax.axis_index("s") == 0)
        def _():
            pltpu.sync_copy(acc0, out0.at[c])
            pltpu.sync_copy(acc1, out1.at[c])

    f = pl.kernel(
        body, mesh=_mesh,
        out_type=[jax.ShapeDtypeStruct((NC, NPAD), jnp.float32)] * 2,
        scratch_types=[
            pltpu.VMEM((CH * 128,), jnp.int32),
            pltpu.VMEM((CH * 128,), jnp.int32),
            pltpu.VMEM((CH * 128,), jnp.float32),
            pltpu.VMEM((CH * 128,), jnp.float32),
            pltpu.VMEM_SHARED((NPAD,), jnp.float32),
            pltpu.VMEM_SHARED((NPAD,), jnp.float32),
        ],
    )
    return f(src1d, dst1d, tab0, tab1, zeros)


def _sc_agg1(src1d, dst1d, tab, zeros):
    def body(src_hbm, dst_hbm, t_hbm, zeros_hbm, out0, idx_s, idx_d, v0, acc0):
        c = lax.axis_index("c")
        w = _wid()
        _zero_acc(zeros_hbm, acc0)
        plsc.subcore_barrier()

        def step(i, carry):
            eb = (w * CPT + i) * CH * 128
            pltpu.sync_copy(src_hbm.at[pl.ds(eb, CH * 128)], idx_s)
            pltpu.sync_copy(dst_hbm.at[pl.ds(eb, CH * 128)], idx_d)
            pltpu.sync_copy(t_hbm.at[idx_s], v0)
            pltpu.sync_copy(v0, acc0.at[idx_d], add=True)
            return carry

        lax.fori_loop(0, CPT, step, 0)
        plsc.subcore_barrier()

        @pl.when(lax.axis_index("s") == 0)
        def _():
            pltpu.sync_copy(acc0, out0.at[c])

    f = pl.kernel(
        body, mesh=_mesh,
        out_type=jax.ShapeDtypeStruct((NC, NPAD), jnp.float32),
        scratch_types=[
            pltpu.VMEM((CH * 128,), jnp.int32),
            pltpu.VMEM((CH * 128,), jnp.int32),
            pltpu.VMEM((CH * 128,), jnp.float32),
            pltpu.VMEM_SHARED((NPAD,), jnp.float32),
        ],
    )
    return f(src1d, dst1d, tab, zeros)


def _tc_prep(degp, x0, x1):
    def body(degp_ref, x0_ref, x1_ref, dis_ref, g0_ref, g1_ref):
        deg = degp_ref[0] + degp_ref[1] + 1.0
        dis = lax.rsqrt(deg)
        dis_ref[...] = dis
        g0_ref[...] = dis * x0_ref[...]
        g1_ref[...] = dis * x1_ref[...]

    return pl.pallas_call(
        body,
        out_shape=[jax.ShapeDtypeStruct((ROWS, 128), jnp.float32)] * 3,
    )(degp, x0, x1)


def _tc_mid(dis, g0, g1, ap0, ap1, W1, b1, W2):
    def body(dis_ref, g0_ref, g1_ref, ap0_ref, ap1_ref, w1_ref, b1_ref,
             w2_ref, z_ref):
        dis = dis_ref[...]
        a0 = dis * (ap0_ref[0] + ap0_ref[1] + g0_ref[...])
        a1 = dis * (ap1_ref[0] + ap1_ref[1] + g1_ref[...])
        z = jnp.zeros_like(dis)
        for k in range(16):
            h = jnp.maximum(a0 * w1_ref[0, k] + a1 * w1_ref[1, k] + b1_ref[k],
                            0.0)
            z = z + h * w2_ref[k, 0]
        z_ref[...] = dis * z

    return pl.pallas_call(
        body,
        in_specs=[
            pl.BlockSpec(memory_space=pltpu.VMEM),
            pl.BlockSpec(memory_space=pltpu.VMEM),
            pl.BlockSpec(memory_space=pltpu.VMEM),
            pl.BlockSpec(memory_space=pltpu.VMEM),
            pl.BlockSpec(memory_space=pltpu.VMEM),
            pl.BlockSpec(memory_space=pltpu.SMEM),
            pl.BlockSpec(memory_space=pltpu.SMEM),
            pl.BlockSpec(memory_space=pltpu.SMEM),
        ],
        out_shape=jax.ShapeDtypeStruct((ROWS, 128), jnp.float32),
    )(dis, g0, g1, ap0, ap1, W1, b1, W2)


def _tc_fin(dis, z, accz, b2):
    def body(dis_ref, z_ref, ap_ref, b2_ref, out_ref):
        out_ref[...] = (dis_ref[...] * (ap_ref[0] + ap_ref[1] + z_ref[...])
                        + b2_ref[0])

    return pl.pallas_call(
        body,
        in_specs=[
            pl.BlockSpec(memory_space=pltpu.VMEM),
            pl.BlockSpec(memory_space=pltpu.VMEM),
            pl.BlockSpec(memory_space=pltpu.VMEM),
            pl.BlockSpec(memory_space=pltpu.SMEM),
        ],
        out_shape=jax.ShapeDtypeStruct((ROWS, 128), jnp.float32),
    )(dis, z, accz, b2)


def kernel(x, edge_index, W1, b1, W2, b2):
    ei = edge_index.astype(jnp.int32)
    pad = jnp.full((EPAD - N_EDGES,), NPAD - 1, jnp.int32)
    src1d = jnp.concatenate([ei[0], pad])
    dst1d = jnp.concatenate([ei[1], pad])

    zeros = jnp.zeros((NPAD,), jnp.float32)
    ones = jnp.ones((CH * 128,), jnp.float32)

    xp = jnp.pad(x, ((0, NPAD - N_NODES), (0, 0)))
    x0 = xp[:, 0].reshape(ROWS, 128)
    x1 = xp[:, 1].reshape(ROWS, 128)

    degp = _sc_deg(dst1d, ones, zeros)
    dis, g0, g1 = _tc_prep(degp.reshape(NC, ROWS, 128), x0, x1)
    ap0, ap1 = _sc_agg2(src1d, dst1d, g0.reshape(NPAD), g1.reshape(NPAD),
                        zeros)
    z = _tc_mid(dis, g0, g1, ap0.reshape(NC, ROWS, 128),
                ap1.reshape(NC, ROWS, 128), W1, b1, W2)
    accz = _sc_agg1(src1d, dst1d, z.reshape(NPAD), zeros)
    out = _tc_fin(dis, z, accz.reshape(NC, ROWS, 128), b2)
    return out.reshape(NPAD)[:N_NODES]


# speedup vs baseline: 172.1199x; 1.5159x over previous
import jax
import jax.numpy as jnp
from jax import lax
from jax.experimental import pallas as pl
from jax.experimental.pallas import tpu as pltpu
from jax.experimental.pallas import tpu_sc as plsc

N_NODES = 100000
N_EDGES = 6400000
NPAD = 102400
ROWS = NPAD // 128
NC, NS = 2, 16
NW = NC * NS
CH = 64
CPT = 25
EPAD = NW * CPT * CH * 128
SLICE = NPAD // NS

_mesh = plsc.VectorSubcoreMesh(core_axis_name="c", subcore_axis_name="s")


def _wid():
    return lax.axis_index("c") * NS + lax.axis_index("s")


def _stage(src_hbm, dst_sh):
    s = lax.axis_index("s")
    pltpu.sync_copy(src_hbm.at[pl.ds(s * SLICE, SLICE)],
                    dst_sh.at[pl.ds(s * SLICE, SLICE)])


def _edge_loop(w, src_hbm, dst_hbm, tabs, idx_s, idx_d, vals, accs,
               sem_i, sem_g, sem_s):
    gather = src_hbm is not None
    C = CH * 128

    def eb(i):
        return (w * CPT + i) * C

    def start_idx(i, p):
        if gather:
            pltpu.async_copy(src_hbm.at[pl.ds(eb(i), C)], idx_s[p], sem_i[p])
        pltpu.async_copy(dst_hbm.at[pl.ds(eb(i), C)], idx_d[p], sem_i[p])

    def wait_idx(p):
        if gather:
            pltpu.make_async_copy(src_hbm.at[pl.ds(0, C)], idx_s[p],
                                  sem_i[p]).wait()
        pltpu.make_async_copy(dst_hbm.at[pl.ds(0, C)], idx_d[p],
                              sem_i[p]).wait()

    def start_gather(p):
        for t, v in zip(tabs, vals):
            pltpu.async_copy(t.at[idx_s[p]], v[p], sem_g[p])

    def wait_gather(p):
        for t, v in zip(tabs, vals):
            pltpu.make_async_copy(t.at[idx_s[p]], v[p], sem_g[p]).wait()

    def start_scat(p):
        for a, v in zip(accs, vals):
            pltpu.async_copy(v[p], a.at[idx_d[p]], sem_s[p], add=True)

    def wait_scat(p):
        for a, v in zip(accs, vals):
            pltpu.make_async_copy(v[p], a.at[idx_d[p]], sem_s[p]).wait()

    def process(i, p, first):
        q = 1 - p
        wait_idx(p)
        if gather:
            start_gather(p)
        if not first:
            wait_scat(q)
        start_idx(jnp.minimum(i + 1, CPT - 1), q)
        if gather:
            wait_gather(p)
        start_scat(p)

    start_idx(0, 0)
    process(0, 0, True)

    def pair(k, carry):
        a = 2 * k + 1
        process(a, 1, False)
        process(a + 1, 0, False)
        return carry

    lax.fori_loop(0, (CPT - 1) // 2, pair, 0)
    wait_idx(1)
    wait_scat(0)


_IDX = pltpu.VMEM((CH * 128,), jnp.int32)
_VAL = pltpu.VMEM((CH * 128,), jnp.float32)
_ACC = pltpu.VMEM_SHARED((NPAD,), jnp.float32)
_SEM = pltpu.SemaphoreType.DMA


def _sc_deg(dst1d, ones, zeros):
    def body(dst_hbm, ones_hbm, zeros_hbm, out_hbm,
             id0, id1, vones, acc,
             si0, si1, ss0, ss1):
        c = lax.axis_index("c")
        pltpu.sync_copy(ones_hbm, vones)
        _stage(zeros_hbm, acc)
        plsc.subcore_barrier()
        _edge_loop(_wid(), None, dst_hbm, [], None, [id0, id1],
                   [[vones, vones]], [acc],
                   [si0, si1], None, [ss0, ss1])
        plsc.subcore_barrier()

        @pl.when(lax.axis_index("s") == 0)
        def _():
            pltpu.sync_copy(acc, out_hbm.at[c])

    f = pl.kernel(
        body, mesh=_mesh,
        out_type=jax.ShapeDtypeStruct((NC, NPAD), jnp.float32),
        scratch_types=[_IDX, _IDX, _VAL, _ACC, _SEM, _SEM, _SEM, _SEM],
    )
    return f(dst1d, ones, zeros)


def _sc_agg2(src1d, dst1d, t0, t1, zeros):
    def body(src_hbm, dst_hbm, t0_hbm, t1_hbm, zeros_hbm, out0, out1,
             is0, is1, id0, id1, v00, v01, v10, v11,
             acc0, acc1, tab0, tab1,
             si0, si1, sg0, sg1, ss0, ss1):
        c = lax.axis_index("c")
        _stage(zeros_hbm, acc0)
        _stage(zeros_hbm, acc1)
        _stage(t0_hbm, tab0)
        _stage(t1_hbm, tab1)
        plsc.subcore_barrier()
        _edge_loop(_wid(), src_hbm, dst_hbm, [tab0, tab1],
                   [is0, is1], [id0, id1],
                   [[v00, v01], [v10, v11]], [acc0, acc1],
                   [si0, si1], [sg0, sg1], [ss0, ss1])
        plsc.subcore_barrier()

        @pl.when(lax.axis_index("s") == 0)
        def _():
            pltpu.sync_copy(acc0, out0.at[c])
            pltpu.sync_copy(acc1, out1.at[c])

    f = pl.kernel(
        body, mesh=_mesh,
        out_type=[jax.ShapeDtypeStruct((NC, NPAD), jnp.float32)] * 2,
        scratch_types=[_IDX, _IDX, _IDX, _IDX, _VAL, _VAL, _VAL, _VAL,
                       _ACC, _ACC, _ACC, _ACC,
                       _SEM, _SEM, _SEM, _SEM, _SEM, _SEM],
    )
    return f(src1d, dst1d, t0, t1, zeros)


def _sc_agg1(src1d, dst1d, tab, zeros):
    def body(src_hbm, dst_hbm, t_hbm, zeros_hbm, out0,
             is0, is1, id0, id1, v0, v1, acc0, tab0,
             si0, si1, sg0, sg1, ss0, ss1):
        c = lax.axis_index("c")
        _stage(zeros_hbm, acc0)
        _stage(t_hbm, tab0)
        plsc.subcore_barrier()
        _edge_loop(_wid(), src_hbm, dst_hbm, [tab0],
                   [is0, is1], [id0, id1],
                   [[v0, v1]], [acc0],
                   [si0, si1], [sg0, sg1], [ss0, ss1])
        plsc.subcore_barrier()

        @pl.when(lax.axis_index("s") == 0)
        def _():
            pltpu.sync_copy(acc0, out0.at[c])

    f = pl.kernel(
        body, mesh=_mesh,
        out_type=jax.ShapeDtypeStruct((NC, NPAD), jnp.float32),
        scratch_types=[_IDX, _IDX, _IDX, _IDX, _VAL, _VAL, _ACC, _ACC,
                       _SEM, _SEM, _SEM, _SEM, _SEM, _SEM],
    )
    return f(src1d, dst1d, tab, zeros)


def _tc_prep(degp, x0, x1):
    def body(degp_ref, x0_ref, x1_ref, dis_ref, g0_ref, g1_ref):
        deg = degp_ref[0] + degp_ref[1] + 1.0
        dis = lax.rsqrt(deg)
        dis_ref[...] = dis
        g0_ref[...] = dis * x0_ref[...]
        g1_ref[...] = dis * x1_ref[...]

    return pl.pallas_call(
        body,
        out_shape=[jax.ShapeDtypeStruct((ROWS, 128), jnp.float32)] * 3,
    )(degp, x0, x1)


def _tc_mid(dis, g0, g1, ap0, ap1, W1, b1, W2):
    def body(dis_ref, g0_ref, g1_ref, ap0_ref, ap1_ref, w1_ref, b1_ref,
             w2_ref, z_ref):
        dis = dis_ref[...]
        a0 = dis * (ap0_ref[0] + ap0_ref[1] + g0_ref[...])
        a1 = dis * (ap1_ref[0] + ap1_ref[1] + g1_ref[...])
        z = jnp.zeros_like(dis)
        for k in range(16):
            h = jnp.maximum(a0 * w1_ref[0, k] + a1 * w1_ref[1, k] + b1_ref[k],
                            0.0)
            z = z + h * w2_ref[k, 0]
        z_ref[...] = dis * z

    return pl.pallas_call(
        body,
        in_specs=[
            pl.BlockSpec(memory_space=pltpu.VMEM),
            pl.BlockSpec(memory_space=pltpu.VMEM),
            pl.BlockSpec(memory_space=pltpu.VMEM),
            pl.BlockSpec(memory_space=pltpu.VMEM),
            pl.BlockSpec(memory_space=pltpu.VMEM),
            pl.BlockSpec(memory_space=pltpu.SMEM),
            pl.BlockSpec(memory_space=pltpu.SMEM),
            pl.BlockSpec(memory_space=pltpu.SMEM),
        ],
        out_shape=jax.ShapeDtypeStruct((ROWS, 128), jnp.float32),
    )(dis, g0, g1, ap0, ap1, W1, b1, W2)


def _tc_fin(dis, z, accz, b2):
    def body(dis_ref, z_ref, ap_ref, b2_ref, out_ref):
        out_ref[...] = (dis_ref[...] * (ap_ref[0] + ap_ref[1] + z_ref[...])
                        + b2_ref[0])

    return pl.pallas_call(
        body,
        in_specs=[
            pl.BlockSpec(memory_space=pltpu.VMEM),
            pl.BlockSpec(memory_space=pltpu.VMEM),
            pl.BlockSpec(memory_space=pltpu.VMEM),
            pl.BlockSpec(memory_space=pltpu.SMEM),
        ],
        out_shape=jax.ShapeDtypeStruct((ROWS, 128), jnp.float32),
    )(dis, z, accz, b2)


def kernel(x, edge_index, W1, b1, W2, b2):
    ei = edge_index.astype(jnp.int32)
    pad = jnp.full((EPAD - N_EDGES,), NPAD - 1, jnp.int32)
    src1d = jnp.concatenate([ei[0], pad])
    dst1d = jnp.concatenate([ei[1], pad])

    zeros = jnp.zeros((NPAD,), jnp.float32)
    ones = jnp.ones((CH * 128,), jnp.float32)

    xp = jnp.pad(x, ((0, NPAD - N_NODES), (0, 0)))
    x0 = xp[:, 0].reshape(ROWS, 128)
    x1 = xp[:, 1].reshape(ROWS, 128)

    degp = _sc_deg(dst1d, ones, zeros)
    dis, g0, g1 = _tc_prep(degp.reshape(NC, ROWS, 128), x0, x1)
    ap0, ap1 = _sc_agg2(src1d, dst1d, g0.reshape(NPAD), g1.reshape(NPAD),
                        zeros)
    z = _tc_mid(dis, g0, g1, ap0.reshape(NC, ROWS, 128),
                ap1.reshape(NC, ROWS, 128), W1, b1, W2)
    accz = _sc_agg1(src1d, dst1d, z.reshape(NPAD), zeros)
    out = _tc_fin(dis, z, accz.reshape(NC, ROWS, 128), b2)
    return out.reshape(NPAD)[:N_NODES]


# speedup vs baseline: 352.8164x; 2.0498x over previous
import jax
import jax.numpy as jnp
from jax import lax
from jax.experimental import pallas as pl
from jax.experimental.pallas import tpu as pltpu
from jax.experimental.pallas import tpu_sc as plsc

N_NODES = 100000
N_EDGES = 6400000
NPAD = 102400
ROWS = NPAD // 128
NC, NS = 2, 16
NW = NC * NS
CHUNK = 8000
CPT = 25
SLICE = NPAD // NS

_mesh = plsc.VectorSubcoreMesh(core_axis_name="c", subcore_axis_name="s")


def _wid():
    return lax.axis_index("c") * NS + lax.axis_index("s")


def _stage(src_hbm, dst_sh):
    s = lax.axis_index("s")
    pltpu.sync_copy(src_hbm.at[pl.ds(s * SLICE, SLICE)],
                    dst_sh.at[pl.ds(s * SLICE, SLICE)])


def _edge_loop(w, src_hbm, dst_hbm, tabs, idx_s, idx_d, vals, accs,
               sem_i, sem_g, sem_s):
    gather = src_hbm is not None
    C = CHUNK

    def eb(i):
        return (w * CPT + i) * C

    def start_idx(i, p):
        if gather:
            pltpu.async_copy(src_hbm.at[pl.ds(eb(i), C)], idx_s[p], sem_i[p])
        pltpu.async_copy(dst_hbm.at[pl.ds(eb(i), C)], idx_d[p], sem_i[p])

    def wait_idx(p):
        if gather:
            pltpu.make_async_copy(src_hbm.at[pl.ds(0, C)], idx_s[p],
                                  sem_i[p]).wait()
        pltpu.make_async_copy(dst_hbm.at[pl.ds(0, C)], idx_d[p],
                              sem_i[p]).wait()

    def start_gather(p):
        for t, v in zip(tabs, vals):
            pltpu.async_copy(t.at[idx_s[p]], v[p], sem_g[p])

    def wait_gather(p):
        for t, v in zip(tabs, vals):
            pltpu.make_async_copy(t.at[idx_s[p]], v[p], sem_g[p]).wait()

    def start_scat(p):
        for a, v in zip(accs, vals):
            pltpu.async_copy(v[p], a.at[idx_d[p]], sem_s[p], add=True)

    def wait_scat(p):
        for a, v in zip(accs, vals):
            pltpu.make_async_copy(v[p], a.at[idx_d[p]], sem_s[p]).wait()

    def process(i, p, first):
        q = 1 - p
        wait_idx(p)
        if gather:
            start_gather(p)
        if not first:
            wait_scat(q)
        start_idx(jnp.minimum(i + 1, CPT - 1), q)
        if gather:
            wait_gather(p)
        start_scat(p)

    start_idx(0, 0)
    process(0, 0, True)

    def pair(k, carry):
        a = 2 * k + 1
        process(a, 1, False)
        process(a + 1, 0, False)
        return carry

    lax.fori_loop(0, (CPT - 1) // 2, pair, 0)
    wait_idx(1)
    wait_scat(0)


_IDX = pltpu.VMEM((CHUNK,), jnp.int32)
_VAL = pltpu.VMEM((CHUNK,), jnp.float32)
_ACC = pltpu.VMEM_SHARED((NPAD,), jnp.float32)
_SEM = pltpu.SemaphoreType.DMA


def _sc_deg(dst1d, ones, zeros):
    def body(dst_hbm, ones_hbm, zeros_hbm, out_hbm,
             id0, id1, vones, acc,
             si0, si1, ss0, ss1):
        c = lax.axis_index("c")
        pltpu.sync_copy(ones_hbm, vones)
        _stage(zeros_hbm, acc)
        plsc.subcore_barrier()
        _edge_loop(_wid(), None, dst_hbm, [], None, [id0, id1],
                   [[vones, vones]], [acc],
                   [si0, si1], None, [ss0, ss1])
        plsc.subcore_barrier()

        @pl.when(lax.axis_index("s") == 0)
        def _():
            pltpu.sync_copy(acc, out_hbm.at[c])

    f = pl.kernel(
        body, mesh=_mesh,
        out_type=jax.ShapeDtypeStruct((NC, NPAD), jnp.float32),
        scratch_types=[_IDX, _IDX, _VAL, _ACC, _SEM, _SEM, _SEM, _SEM],
    )
    return f(dst1d, ones, zeros)


def _sc_agg2(src1d, dst1d, t0, t1, zeros):
    def body(src_hbm, dst_hbm, t0_hbm, t1_hbm, zeros_hbm, out0, out1,
             is0, is1, id0, id1, v00, v01, v10, v11,
             acc0, acc1, tab0, tab1,
             si0, si1, sg0, sg1, ss0, ss1):
        c = lax.axis_index("c")
        _stage(zeros_hbm, acc0)
        _stage(zeros_hbm, acc1)
        _stage(t0_hbm, tab0)
        _stage(t1_hbm, tab1)
        plsc.subcore_barrier()
        _edge_loop(_wid(), src_hbm, dst_hbm, [tab0, tab1],
                   [is0, is1], [id0, id1],
                   [[v00, v01], [v10, v11]], [acc0, acc1],
                   [si0, si1], [sg0, sg1], [ss0, ss1])
        plsc.subcore_barrier()

        @pl.when(lax.axis_index("s") == 0)
        def _():
            pltpu.sync_copy(acc0, out0.at[c])
            pltpu.sync_copy(acc1, out1.at[c])

    f = pl.kernel(
        body, mesh=_mesh,
        out_type=[jax.ShapeDtypeStruct((NC, NPAD), jnp.float32)] * 2,
        scratch_types=[_IDX, _IDX, _IDX, _IDX, _VAL, _VAL, _VAL, _VAL,
                       _ACC, _ACC, _ACC, _ACC,
                       _SEM, _SEM, _SEM, _SEM, _SEM, _SEM],
    )
    return f(src1d, dst1d, t0, t1, zeros)


def _sc_agg1(src1d, dst1d, tab, zeros):
    def body(src_hbm, dst_hbm, t_hbm, zeros_hbm, out0,
             is0, is1, id0, id1, v0, v1, acc0, tab0,
             si0, si1, sg0, sg1, ss0, ss1):
        c = lax.axis_index("c")
        _stage(zeros_hbm, acc0)
        _stage(t_hbm, tab0)
        plsc.subcore_barrier()
        _edge_loop(_wid(), src_hbm, dst_hbm, [tab0],
                   [is0, is1], [id0, id1],
                   [[v0, v1]], [acc0],
                   [si0, si1], [sg0, sg1], [ss0, ss1])
        plsc.subcore_barrier()

        @pl.when(lax.axis_index("s") == 0)
        def _():
            pltpu.sync_copy(acc0, out0.at[c])

    f = pl.kernel(
        body, mesh=_mesh,
        out_type=jax.ShapeDtypeStruct((NC, NPAD), jnp.float32),
        scratch_types=[_IDX, _IDX, _IDX, _IDX, _VAL, _VAL, _ACC, _ACC,
                       _SEM, _SEM, _SEM, _SEM, _SEM, _SEM],
    )
    return f(src1d, dst1d, tab, zeros)


def _tc_prep(degp, x0, x1):
    def body(degp_ref, x0_ref, x1_ref, dis_ref, g0_ref, g1_ref):
        deg = degp_ref[0] + degp_ref[1] + 1.0
        dis = lax.rsqrt(deg)
        dis_ref[...] = dis
        g0_ref[...] = dis * x0_ref[...]
        g1_ref[...] = dis * x1_ref[...]

    return pl.pallas_call(
        body,
        out_shape=[jax.ShapeDtypeStruct((ROWS, 128), jnp.float32)] * 3,
    )(degp, x0, x1)


def _tc_mid(dis, g0, g1, ap0, ap1, W1, b1, W2):
    def body(dis_ref, g0_ref, g1_ref, ap0_ref, ap1_ref, w1_ref, b1_ref,
             w2_ref, z_ref):
        dis = dis_ref[...]
        a0 = dis * (ap0_ref[0] + ap0_ref[1] + g0_ref[...])
        a1 = dis * (ap1_ref[0] + ap1_ref[1] + g1_ref[...])
        z = jnp.zeros_like(dis)
        for k in range(16):
            h = jnp.maximum(a0 * w1_ref[0, k] + a1 * w1_ref[1, k] + b1_ref[k],
                            0.0)
            z = z + h * w2_ref[k, 0]
        z_ref[...] = dis * z

    return pl.pallas_call(
        body,
        in_specs=[
            pl.BlockSpec(memory_space=pltpu.VMEM),
            pl.BlockSpec(memory_space=pltpu.VMEM),
            pl.BlockSpec(memory_space=pltpu.VMEM),
            pl.BlockSpec(memory_space=pltpu.VMEM),
            pl.BlockSpec(memory_space=pltpu.VMEM),
            pl.BlockSpec(memory_space=pltpu.SMEM),
            pl.BlockSpec(memory_space=pltpu.SMEM),
            pl.BlockSpec(memory_space=pltpu.SMEM),
        ],
        out_shape=jax.ShapeDtypeStruct((ROWS, 128), jnp.float32),
    )(dis, g0, g1, ap0, ap1, W1, b1, W2)


def _tc_fin(dis, z, accz, b2):
    def body(dis_ref, z_ref, ap_ref, b2_ref, out_ref):
        out_ref[...] = (dis_ref[...] * (ap_ref[0] + ap_ref[1] + z_ref[...])
                        + b2_ref[0])

    return pl.pallas_call(
        body,
        in_specs=[
            pl.BlockSpec(memory_space=pltpu.VMEM),
            pl.BlockSpec(memory_space=pltpu.VMEM),
            pl.BlockSpec(memory_space=pltpu.VMEM),
            pl.BlockSpec(memory_space=pltpu.SMEM),
        ],
        out_shape=jax.ShapeDtypeStruct((ROWS, 128), jnp.float32),
    )(dis, z, accz, b2)


def kernel(x, edge_index, W1, b1, W2, b2):
    ei = edge_index.astype(jnp.int32)
    src1d = ei[0]
    dst1d = ei[1]

    zeros = jnp.zeros((NPAD,), jnp.float32)
    ones = jnp.ones((CHUNK,), jnp.float32)

    xp = jnp.pad(x, ((0, NPAD - N_NODES), (0, 0)))
    x0 = xp[:, 0].reshape(ROWS, 128)
    x1 = xp[:, 1].reshape(ROWS, 128)

    degp = _sc_deg(dst1d, ones, zeros)
    dis, g0, g1 = _tc_prep(degp.reshape(NC, ROWS, 128), x0, x1)
    ap0, ap1 = _sc_agg2(src1d, dst1d, g0.reshape(NPAD), g1.reshape(NPAD),
                        zeros)
    z = _tc_mid(dis, g0, g1, ap0.reshape(NC, ROWS, 128),
                ap1.reshape(NC, ROWS, 128), W1, b1, W2)
    accz = _sc_agg1(src1d, dst1d, z.reshape(NPAD), zeros)
    out = _tc_fin(dis, z, accz.reshape(NC, ROWS, 128), b2)
    return out.reshape(NPAD)[:N_NODES]
